# baseline (speedup 1.0000x reference)
import functools

import jax
import jax.numpy as jnp
from jax import lax
from jax.experimental import pallas as pl
from jax.experimental.pallas import tpu as pltpu
from jax.experimental.pallas import tpu_sc as plsc

VOCAB = 100000
D = 300
B = 4096 * 50
NC = 2
NS = 16
NW = NC * NS
BPW = B // NW
CHUNK = 64
NCHUNK = BPW // CHUNK
GRAN = 320
NGRAN = VOCAB * D // GRAN
G0MAX = NGRAN - 2

_MESH = plsc.VectorSubcoreMesh(core_axis_name="c", subcore_axis_name="s")


@functools.partial(
    pl.kernel,
    mesh=_MESH,
    out_type=jax.ShapeDtypeStruct((B * D,), jnp.float32),
    compiler_params=pltpu.CompilerParams(
        use_tc_tiling_on_sc=False, needs_layout_passes=False),
    scratch_types=[
        pltpu.VMEM((NCHUNK, 2 * CHUNK), jnp.int32),
        pltpu.VMEM((BPW,), jnp.int32),
        pltpu.VMEM((2 * CHUNK, GRAN), jnp.float32),
        pltpu.VMEM((2 * CHUNK, GRAN), jnp.float32),
        pltpu.VMEM((CHUNK * D + 16,), jnp.float32),
        pltpu.SemaphoreType.DMA,
        pltpu.SemaphoreType.DMA,
    ],
)
def _gather_kernel(gidx_hbm, ph_hbm, tview_hbm, out_hbm, gidx_v, ph_v,
                   buf0, buf1, cmp_v, sem0, sem1):
    wid = lax.axis_index("s") * NC + lax.axis_index("c")
    base = wid * BPW
    pltpu.sync_copy(gidx_hbm.at[wid], gidx_v)
    pltpu.sync_copy(ph_hbm.at[wid], ph_v)

    lane = lax.iota(jnp.int32, 16)
    bufs = (buf0, buf1)
    sems = (sem0, sem1)

    def fire(c, par):
        pltpu.async_copy(
            tview_hbm.at[gidx_v.at[c]], bufs[par], sems[par])

    def wait(par):
        pltpu.make_async_copy(
            tview_hbm.at[gidx_v.at[0]], bufs[par], sems[par]).wait()

    def realign(c, buf_v):
        def row_body(r, carry):
            grp16 = lax.shift_right_logical(r, 4) * 16
            l = lax.bitwise_and(r, 15)
            pv = ph_v[pl.ds(c * CHUNK + grp16, 16)]
            p = lax.reduce_max(jnp.where(lane == l, pv, 0), (0,))
            qbase = p + lane
            slot0 = r * 2
            dst0 = r * D
            for k in range(D // 16 + 1):
                q = qbase + 16 * k
                ge = q >= GRAN
                slot = slot0 + ge.astype(jnp.int32)
                w = jnp.where(ge, q - GRAN, q)
                cmp_v[pl.ds(dst0 + 16 * k, 16)] = plsc.load_gather(
                    buf_v, [slot, w])
            return carry

        lax.fori_loop(0, CHUNK, row_body, 0)
        pltpu.sync_copy(
            cmp_v.at[pl.ds(0, CHUNK * D)],
            out_hbm.at[pl.ds((base + c * CHUNK) * D, CHUNK * D)])

    fire(0, 0)

    def chunk_body(c, carry):
        for par in range(2):
            @pl.when(lax.rem(c, 2) == par)
            def _():
                @pl.when(c + 1 < NCHUNK)
                def _():
                    fire(c + 1, 1 - par)
                wait(par)
                realign(c, bufs[par])
        return carry

    lax.fori_loop(0, NCHUNK, chunk_body, 0)


def kernel(inp, table):
    idx = inp.reshape(-1).astype(jnp.int32)
    q = idx * D
    g0 = jnp.minimum(q // GRAN, G0MAX)
    ph = (q - g0 * GRAN).reshape(NW, BPW)
    gidx = jnp.stack([g0, g0 + 1], axis=-1).reshape(NW, NCHUNK, 2 * CHUNK)
    tview = table.reshape(NGRAN, GRAN)
    out = _gather_kernel(gidx, ph, tview)
    return out.reshape(inp.shape[0], inp.shape[1], D)
